# baseline (speedup 1.0000x reference)
import functools

import jax
import jax.numpy as jnp
from jax import lax
from jax.experimental import pallas as pl
from jax.experimental.pallas import tpu as pltpu
from jax.experimental.pallas import tpu_sc as plsc

B, T = 1024, 50
V1, D, H, O = 100001, 300, 128, 4


_BV = 2048


def _proj_body(emb_ref, wf_ref, bf_ref, wb_ref, bb_ref, pf_ref, pb_ref):
    e = emb_ref[...].astype(jnp.bfloat16)
    wf = wf_ref[...].astype(jnp.bfloat16)
    wb = wb_ref[...].astype(jnp.bfloat16)
    pf_ref[...] = lax.dot_general(e, wf, (((1,), (1,)), ((), ())),
                                  preferred_element_type=jnp.float32) + bf_ref[...]
    pb_ref[...] = lax.dot_general(e, wb, (((1,), (1,)), ((), ())),
                                  preferred_element_type=jnp.float32) + bb_ref[...]


def _tc_project(emb, W_ih_f, bf, W_ih_b, bb):
    nv = pl.cdiv(V1, _BV)
    return pl.pallas_call(
        _proj_body,
        grid=(nv,),
        in_specs=[
            pl.BlockSpec((_BV, D), lambda i: (i, 0)),
            pl.BlockSpec((H, D), lambda i: (0, 0)),
            pl.BlockSpec((1, H), lambda i: (0, 0)),
            pl.BlockSpec((H, D), lambda i: (0, 0)),
            pl.BlockSpec((1, H), lambda i: (0, 0)),
        ],
        out_specs=[
            pl.BlockSpec((_BV, H), lambda i: (i, 0)),
            pl.BlockSpec((_BV, H), lambda i: (i, 0)),
        ],
        out_shape=[
            jax.ShapeDtypeStruct((V1, H), jnp.float32),
            jax.ShapeDtypeStruct((V1, H), jnp.float32),
        ],
        compiler_params=pltpu.CompilerParams(
            dimension_semantics=("parallel",)),
    )(emb, W_ih_f, bf, W_ih_b, bb)


_N = B * T
_CHUNK = 200


def _sc_gather(pf, pb, idx, idx_last):
    info = plsc.get_sparse_core_info()
    nw = info.num_cores * info.num_subcores
    per_w = _N // nw
    nch = per_w // _CHUNK
    per_w_l = B // nw
    mesh = plsc.VectorSubcoreMesh(core_axis_name="c", subcore_axis_name="s")

    @functools.partial(
        pl.kernel,
        mesh=mesh,
        out_type=[
            jax.ShapeDtypeStruct((_N, H), jnp.float32),
            jax.ShapeDtypeStruct((B, H), jnp.float32),
        ],
        scratch_types=[
            pltpu.VMEM((_CHUNK,), jnp.int32),
            pltpu.VMEM((_CHUNK, H), jnp.float32),
            pltpu.VMEM((per_w_l,), jnp.int32),
            pltpu.VMEM((per_w_l, H), jnp.float32),
            pltpu.SemaphoreType.DMA,
        ],
    )
    def k(pf_hbm, pb_hbm, idx_hbm, idxl_hbm, uf_hbm, ub_hbm,
          idx_v, rows_v, idxl_v, rowsl_v, sem):
        wid = lax.axis_index("s") * info.num_cores + lax.axis_index("c")
        base = wid * per_w
        for c in range(nch):
            off = base + c * _CHUNK
            pltpu.sync_copy(idx_hbm.at[pl.ds(off, _CHUNK)], idx_v)
            pltpu.async_copy(pf_hbm.at[idx_v], rows_v, sem).wait()
            pltpu.sync_copy(rows_v, uf_hbm.at[pl.ds(off, _CHUNK)])
        basel = wid * per_w_l
        pltpu.sync_copy(idxl_hbm.at[pl.ds(basel, per_w_l)], idxl_v)
        pltpu.async_copy(pb_hbm.at[idxl_v], rowsl_v, sem).wait()
        pltpu.sync_copy(rowsl_v, ub_hbm.at[pl.ds(basel, per_w_l)])

    return k(pf, pb, idx, idx_last)


BB = 512
TT = 10
NB = B // BB
NT = T // TT


def _rnn_body(uf_ref, Whh_ref, ub_ref, Wfcf_ref, Wfcb_ref, bfc_ref,
              out_ref, h_ref):
    t = pl.program_id(1)

    @pl.when(t == 0)
    def _():
        h_ref[...] = jnp.zeros_like(h_ref)

    u = uf_ref[...]
    h = h_ref[...]
    Whh = Whh_ref[...]
    for tt in range(TT):
        hh = lax.dot_general(h, Whh, (((1,), (1,)), ((), ())),
                             preferred_element_type=jnp.float32)
        h = jnp.tanh(u[tt] + hh)
    h_ref[...] = h

    @pl.when(t == NT - 1)
    def _():
        hb = jnp.tanh(ub_ref[...])
        logits = (lax.dot_general(h, Wfcf_ref[...], (((1,), (1,)), ((), ())),
                                  preferred_element_type=jnp.float32)
                  + lax.dot_general(hb, Wfcb_ref[...],
                                    (((1,), (1,)), ((), ())),
                                    preferred_element_type=jnp.float32)
                  + bfc_ref[...])
        m = jnp.max(logits, axis=1, keepdims=True)
        e = jnp.exp(logits - m)
        out_ref[...] = e / jnp.sum(e, axis=1, keepdims=True)


def _tc_rnn(uf_tbh, W_hh_f, ub, Wfcf, Wfcb, bfc):
    return pl.pallas_call(
        _rnn_body,
        grid=(NB, NT),
        in_specs=[
            pl.BlockSpec((TT, BB, H), lambda b, t: (t, b, 0)),
            pl.BlockSpec((H, H), lambda b, t: (0, 0)),
            pl.BlockSpec((BB, H), lambda b, t: (b, 0)),
            pl.BlockSpec((O, H), lambda b, t: (0, 0)),
            pl.BlockSpec((O, H), lambda b, t: (0, 0)),
            pl.BlockSpec((1, O), lambda b, t: (0, 0)),
        ],
        out_specs=pl.BlockSpec((BB, O), lambda b, t: (b, 0)),
        out_shape=jax.ShapeDtypeStruct((B, O), jnp.float32),
        scratch_shapes=[pltpu.VMEM((BB, H), jnp.float32)],
        compiler_params=pltpu.CompilerParams(
            dimension_semantics=("parallel", "arbitrary")),
    )(uf_tbh, W_hh_f, ub, Wfcf, Wfcb, bfc)


def kernel(x, emb, W_ih_f, W_hh_f, b_ih_f, b_hh_f,
           W_ih_b, W_hh_b, b_ih_b, b_hh_b, W_fc, b_fc):
    bf = (b_ih_f + b_hh_f).reshape(1, H)
    bb = (b_ih_b + b_hh_b).reshape(1, H)
    pf, pb = _tc_project(emb, W_ih_f, bf, W_ih_b, bb)

    idx = jnp.transpose(x).reshape(-1)
    idx_last = x[:, -1]
    uf, ub = _sc_gather(pf, pb, idx, idx_last)
    uf_tbh = uf.reshape(T, B, H)

    Wfcf = W_fc[:, :H]
    Wfcb = W_fc[:, H:]
    bfc = b_fc.reshape(1, O)

    return _tc_rnn(uf_tbh, W_hh_f, ub, Wfcf, Wfcb, bfc)


# speedup vs baseline: 1.0453x; 1.0453x over previous
import functools

import jax
import jax.numpy as jnp
from jax import lax
from jax.experimental import pallas as pl
from jax.experimental.pallas import tpu as pltpu
from jax.experimental.pallas import tpu_sc as plsc

B, T = 1024, 50
V1, D, H, O = 100001, 300, 128, 4


_BV = 2048


def _proj_body(emb_ref, wf_ref, bf_ref, wb_ref, bb_ref, p_ref):
    e = emb_ref[...]
    pf = lax.dot_general(e, wf_ref[...], (((1,), (1,)), ((), ())),
                         preferred_element_type=jnp.float32) + bf_ref[...]
    pb = lax.dot_general(e, wb_ref[...], (((1,), (1,)), ((), ())),
                         preferred_element_type=jnp.float32) + bb_ref[...]
    pf_bits = lax.bitcast_convert_type(
        pf.astype(jnp.bfloat16).astype(jnp.float32), jnp.uint32)
    pb_bits = lax.bitcast_convert_type(
        pb.astype(jnp.bfloat16).astype(jnp.float32), jnp.uint32)
    packed = (pf_bits >> 16) | pb_bits
    p_ref[...] = lax.bitcast_convert_type(packed, jnp.int32)


def _tc_project(emb, W_ih_f, bf, W_ih_b, bb):
    nv = pl.cdiv(V1, _BV)
    return pl.pallas_call(
        _proj_body,
        grid=(nv,),
        in_specs=[
            pl.BlockSpec((_BV, D), lambda i: (i, 0)),
            pl.BlockSpec((H, D), lambda i: (0, 0)),
            pl.BlockSpec((1, H), lambda i: (0, 0)),
            pl.BlockSpec((H, D), lambda i: (0, 0)),
            pl.BlockSpec((1, H), lambda i: (0, 0)),
        ],
        out_specs=pl.BlockSpec((_BV, H), lambda i: (i, 0)),
        out_shape=jax.ShapeDtypeStruct((V1, H), jnp.int32),
        compiler_params=pltpu.CompilerParams(
            dimension_semantics=("parallel",)),
    )(emb, W_ih_f, bf, W_ih_b, bb)


_N = B * T
_CHUNK = 200


def _sc_gather(p, idx):
    info = plsc.get_sparse_core_info()
    nw = info.num_cores * info.num_subcores
    per_w = _N // nw
    nch = per_w // _CHUNK
    mesh = plsc.VectorSubcoreMesh(core_axis_name="c", subcore_axis_name="s")

    @functools.partial(
        pl.kernel,
        mesh=mesh,
        out_type=jax.ShapeDtypeStruct((_N, H), jnp.int32),
        scratch_types=[
            pltpu.VMEM((_CHUNK,), jnp.int32),
            pltpu.VMEM((_CHUNK, H), jnp.int32),
            pltpu.SemaphoreType.DMA,
        ],
    )
    def k(p_hbm, idx_hbm, u_hbm, idx_v, rows_v, sem):
        wid = lax.axis_index("s") * info.num_cores + lax.axis_index("c")
        base = wid * per_w
        for c in range(nch):
            off = base + c * _CHUNK
            pltpu.sync_copy(idx_hbm.at[pl.ds(off, _CHUNK)], idx_v)
            pltpu.async_copy(p_hbm.at[idx_v], rows_v, sem).wait()
            pltpu.sync_copy(rows_v, u_hbm.at[pl.ds(off, _CHUNK)])

    return k(p, idx)


BB = 512
TT = 10
NB = B // BB
NT = T // TT


def _rnn_body(u_ref, Whh_ref, Wfcf_ref, Wfcb_ref, bfc_ref, out_ref, h_ref):
    t = pl.program_id(1)

    @pl.when(t == 0)
    def _():
        h_ref[...] = jnp.zeros_like(h_ref)

    uu = lax.bitcast_convert_type(u_ref[...], jnp.uint32)
    u = lax.bitcast_convert_type(uu << 16, jnp.float32)
    h = h_ref[...]
    Whh = Whh_ref[...]
    for tt in range(TT):
        hh = lax.dot_general(h, Whh, (((1,), (1,)), ((), ())),
                             preferred_element_type=jnp.float32)
        h = jnp.tanh(u[tt] + hh)
    h_ref[...] = h

    @pl.when(t == NT - 1)
    def _():
        ub = lax.bitcast_convert_type(
            uu[TT - 1] & jnp.uint32(0xFFFF0000), jnp.float32)
        hb = jnp.tanh(ub)
        logits = (lax.dot_general(h, Wfcf_ref[...], (((1,), (1,)), ((), ())),
                                  preferred_element_type=jnp.float32)
                  + lax.dot_general(hb, Wfcb_ref[...],
                                    (((1,), (1,)), ((), ())),
                                    preferred_element_type=jnp.float32)
                  + bfc_ref[...])
        m = jnp.max(logits, axis=1, keepdims=True)
        e = jnp.exp(logits - m)
        out_ref[...] = e / jnp.sum(e, axis=1, keepdims=True)


def _tc_rnn(u_tbh, W_hh_f, Wfcf, Wfcb, bfc):
    return pl.pallas_call(
        _rnn_body,
        grid=(NB, NT),
        in_specs=[
            pl.BlockSpec((TT, BB, H), lambda b, t: (t, b, 0)),
            pl.BlockSpec((H, H), lambda b, t: (0, 0)),
            pl.BlockSpec((O, H), lambda b, t: (0, 0)),
            pl.BlockSpec((O, H), lambda b, t: (0, 0)),
            pl.BlockSpec((1, O), lambda b, t: (0, 0)),
        ],
        out_specs=pl.BlockSpec((BB, O), lambda b, t: (b, 0)),
        out_shape=jax.ShapeDtypeStruct((B, O), jnp.float32),
        scratch_shapes=[pltpu.VMEM((BB, H), jnp.float32)],
        compiler_params=pltpu.CompilerParams(
            dimension_semantics=("parallel", "arbitrary")),
    )(u_tbh, W_hh_f, Wfcf, Wfcb, bfc)


def kernel(x, emb, W_ih_f, W_hh_f, b_ih_f, b_hh_f,
           W_ih_b, W_hh_b, b_ih_b, b_hh_b, W_fc, b_fc):
    bf = (b_ih_f + b_hh_f).reshape(1, H)
    bb = (b_ih_b + b_hh_b).reshape(1, H)
    p = _tc_project(emb, W_ih_f, bf, W_ih_b, bb)

    idx = jnp.transpose(x).reshape(-1)
    u = _sc_gather(p, idx)
    u_tbh = u.reshape(T, B, H)

    Wfcf = W_fc[:, :H]
    Wfcb = W_fc[:, H:]
    bfc = b_fc.reshape(1, O)

    return _tc_rnn(u_tbh, W_hh_f, Wfcf, Wfcb, bfc)


# speedup vs baseline: 1.1043x; 1.0565x over previous
import functools

import jax
import jax.numpy as jnp
from jax import lax
from jax.experimental import pallas as pl
from jax.experimental.pallas import tpu as pltpu
from jax.experimental.pallas import tpu_sc as plsc

B, T = 1024, 50
V1, D, H, O = 100001, 300, 128, 4


_BV = 4096


def _proj_body(emb_ref, wf_ref, bf_ref, wb_ref, bb_ref, p_ref):
    e = emb_ref[...]
    pf = lax.dot_general(e, wf_ref[...], (((1,), (1,)), ((), ())),
                         preferred_element_type=jnp.float32) + bf_ref[...]
    pb = lax.dot_general(e, wb_ref[...], (((1,), (1,)), ((), ())),
                         preferred_element_type=jnp.float32) + bb_ref[...]
    pf_bits = lax.bitcast_convert_type(
        pf.astype(jnp.bfloat16).astype(jnp.float32), jnp.uint32)
    pb_bits = lax.bitcast_convert_type(
        pb.astype(jnp.bfloat16).astype(jnp.float32), jnp.uint32)
    packed = (pf_bits >> 16) | pb_bits
    p_ref[...] = lax.bitcast_convert_type(packed, jnp.int32)


def _tc_project(emb, W_ih_f, bf, W_ih_b, bb):
    nv = pl.cdiv(V1, _BV)
    return pl.pallas_call(
        _proj_body,
        grid=(nv,),
        in_specs=[
            pl.BlockSpec((_BV, D), lambda i: (i, 0)),
            pl.BlockSpec((H, D), lambda i: (0, 0)),
            pl.BlockSpec((1, H), lambda i: (0, 0)),
            pl.BlockSpec((H, D), lambda i: (0, 0)),
            pl.BlockSpec((1, H), lambda i: (0, 0)),
        ],
        out_specs=pl.BlockSpec((_BV, H), lambda i: (i, 0)),
        out_shape=jax.ShapeDtypeStruct((V1, H), jnp.int32),
        compiler_params=pltpu.CompilerParams(
            dimension_semantics=("parallel",)),
    )(emb, W_ih_f, bf, W_ih_b, bb)


_N = B * T
_CHUNK = 200


def _sc_gather(p, idx):
    info = plsc.get_sparse_core_info()
    nw = info.num_cores * info.num_subcores
    per_w = _N // nw
    nch = per_w // _CHUNK
    mesh = plsc.VectorSubcoreMesh(core_axis_name="c", subcore_axis_name="s")

    @functools.partial(
        pl.kernel,
        mesh=mesh,
        out_type=jax.ShapeDtypeStruct((_N, H), jnp.int32),
        scratch_types=[
            pltpu.VMEM((_CHUNK,), jnp.int32),
            pltpu.VMEM((_CHUNK, H), jnp.int32),
            pltpu.SemaphoreType.DMA,
        ],
    )
    def k(p_hbm, idx_hbm, u_hbm, idx_v, rows_v, sem):
        wid = lax.axis_index("s") * info.num_cores + lax.axis_index("c")
        base = wid * per_w
        for c in range(nch):
            off = base + c * _CHUNK
            pltpu.sync_copy(idx_hbm.at[pl.ds(off, _CHUNK)], idx_v)
            pltpu.async_copy(p_hbm.at[idx_v], rows_v, sem).wait()
            pltpu.sync_copy(rows_v, u_hbm.at[pl.ds(off, _CHUNK)])

    return k(p, idx)


BB = 512
TT = 10
NB = B // BB
NT = T // TT


def _rnn_body(u_ref, Whh_ref, Wfcf_ref, Wfcb_ref, bfc_ref, out_ref, h_ref):
    t = pl.program_id(1)

    @pl.when(t == 0)
    def _():
        h_ref[...] = jnp.zeros_like(h_ref)

    uu = lax.bitcast_convert_type(u_ref[...], jnp.uint32)
    u = lax.bitcast_convert_type(uu << 16, jnp.float32)
    h = h_ref[...]
    Whh = Whh_ref[...]
    for tt in range(TT):
        hh = lax.dot_general(h, Whh, (((1,), (1,)), ((), ())),
                             preferred_element_type=jnp.float32)
        h = jnp.tanh(u[tt] + hh)
    h_ref[...] = h

    @pl.when(t == NT - 1)
    def _():
        ub = lax.bitcast_convert_type(
            uu[TT - 1] & jnp.uint32(0xFFFF0000), jnp.float32)
        hb = jnp.tanh(ub)
        logits = (lax.dot_general(h, Wfcf_ref[...], (((1,), (1,)), ((), ())),
                                  preferred_element_type=jnp.float32)
                  + lax.dot_general(hb, Wfcb_ref[...],
                                    (((1,), (1,)), ((), ())),
                                    preferred_element_type=jnp.float32)
                  + bfc_ref[...])
        m = jnp.max(logits, axis=1, keepdims=True)
        e = jnp.exp(logits - m)
        out_ref[...] = e / jnp.sum(e, axis=1, keepdims=True)


def _tc_rnn(u_tbh, W_hh_f, Wfcf, Wfcb, bfc):
    return pl.pallas_call(
        _rnn_body,
        grid=(NB, NT),
        in_specs=[
            pl.BlockSpec((TT, BB, H), lambda b, t: (t, b, 0)),
            pl.BlockSpec((H, H), lambda b, t: (0, 0)),
            pl.BlockSpec((O, H), lambda b, t: (0, 0)),
            pl.BlockSpec((O, H), lambda b, t: (0, 0)),
            pl.BlockSpec((1, O), lambda b, t: (0, 0)),
        ],
        out_specs=pl.BlockSpec((BB, O), lambda b, t: (b, 0)),
        out_shape=jax.ShapeDtypeStruct((B, O), jnp.float32),
        scratch_shapes=[pltpu.VMEM((BB, H), jnp.float32)],
        compiler_params=pltpu.CompilerParams(
            dimension_semantics=("parallel", "arbitrary")),
    )(u_tbh, W_hh_f, Wfcf, Wfcb, bfc)


def kernel(x, emb, W_ih_f, W_hh_f, b_ih_f, b_hh_f,
           W_ih_b, W_hh_b, b_ih_b, b_hh_b, W_fc, b_fc):
    bf = (b_ih_f + b_hh_f).reshape(1, H)
    bb = (b_ih_b + b_hh_b).reshape(1, H)
    p = _tc_project(emb, W_ih_f, bf, W_ih_b, bb)

    idx = jnp.transpose(x).reshape(-1)
    u = _sc_gather(p, idx)
    u_tbh = u.reshape(T, B, H)

    Wfcf = W_fc[:, :H]
    Wfcb = W_fc[:, H:]
    bfc = b_fc.reshape(1, O)

    return _tc_rnn(u_tbh, W_hh_f, Wfcf, Wfcb, bfc)


# speedup vs baseline: 1.1093x; 1.0045x over previous
import functools

import jax
import jax.numpy as jnp
from jax import lax
from jax.experimental import pallas as pl
from jax.experimental.pallas import tpu as pltpu
from jax.experimental.pallas import tpu_sc as plsc

B, T = 1024, 50
V1, D, H, O = 100001, 300, 128, 4


_BV = 8192


def _proj_body(emb_ref, wf_ref, bf_ref, wb_ref, bb_ref, p_ref):
    e = emb_ref[...]
    pf = lax.dot_general(e, wf_ref[...], (((1,), (1,)), ((), ())),
                         preferred_element_type=jnp.float32) + bf_ref[...]
    pb = lax.dot_general(e, wb_ref[...], (((1,), (1,)), ((), ())),
                         preferred_element_type=jnp.float32) + bb_ref[...]
    pf_bits = lax.bitcast_convert_type(
        pf.astype(jnp.bfloat16).astype(jnp.float32), jnp.uint32)
    pb_bits = lax.bitcast_convert_type(
        pb.astype(jnp.bfloat16).astype(jnp.float32), jnp.uint32)
    packed = (pf_bits >> 16) | pb_bits
    p_ref[...] = lax.bitcast_convert_type(packed, jnp.int32)


def _tc_project(emb, W_ih_f, bf, W_ih_b, bb):
    nv = pl.cdiv(V1, _BV)
    return pl.pallas_call(
        _proj_body,
        grid=(nv,),
        in_specs=[
            pl.BlockSpec((_BV, D), lambda i: (i, 0)),
            pl.BlockSpec((H, D), lambda i: (0, 0)),
            pl.BlockSpec((1, H), lambda i: (0, 0)),
            pl.BlockSpec((H, D), lambda i: (0, 0)),
            pl.BlockSpec((1, H), lambda i: (0, 0)),
        ],
        out_specs=pl.BlockSpec((_BV, H), lambda i: (i, 0)),
        out_shape=jax.ShapeDtypeStruct((V1, H), jnp.int32),
        compiler_params=pltpu.CompilerParams(
            dimension_semantics=("parallel",)),
    )(emb, W_ih_f, bf, W_ih_b, bb)


_N = B * T
_CHUNK = 200


def _sc_gather(p, idx):
    info = plsc.get_sparse_core_info()
    nw = info.num_cores * info.num_subcores
    per_w = _N // nw
    nch = per_w // _CHUNK
    mesh = plsc.VectorSubcoreMesh(core_axis_name="c", subcore_axis_name="s")

    @functools.partial(
        pl.kernel,
        mesh=mesh,
        out_type=jax.ShapeDtypeStruct((_N, H), jnp.int32),
        scratch_types=[
            pltpu.VMEM((_CHUNK,), jnp.int32),
            pltpu.VMEM((_CHUNK, H), jnp.int32),
            pltpu.SemaphoreType.DMA,
        ],
    )
    def k(p_hbm, idx_hbm, u_hbm, idx_v, rows_v, sem):
        wid = lax.axis_index("s") * info.num_cores + lax.axis_index("c")
        base = wid * per_w
        for c in range(nch):
            off = base + c * _CHUNK
            pltpu.sync_copy(idx_hbm.at[pl.ds(off, _CHUNK)], idx_v)
            pltpu.async_copy(p_hbm.at[idx_v], rows_v, sem).wait()
            pltpu.sync_copy(rows_v, u_hbm.at[pl.ds(off, _CHUNK)])

    return k(p, idx)


BB = 512
TT = 10
NB = B // BB
NT = T // TT


def _rnn_body(u_ref, Whh_ref, Wfcf_ref, Wfcb_ref, bfc_ref, out_ref, h_ref):
    t = pl.program_id(1)

    @pl.when(t == 0)
    def _():
        h_ref[...] = jnp.zeros_like(h_ref)

    uu = lax.bitcast_convert_type(u_ref[...], jnp.uint32)
    u = lax.bitcast_convert_type(uu << 16, jnp.float32)
    h = h_ref[...]
    Whh = Whh_ref[...]
    for tt in range(TT):
        hh = lax.dot_general(h, Whh, (((1,), (1,)), ((), ())),
                             preferred_element_type=jnp.float32)
        h = jnp.tanh(u[tt] + hh)
    h_ref[...] = h

    @pl.when(t == NT - 1)
    def _():
        ub = lax.bitcast_convert_type(
            uu[TT - 1] & jnp.uint32(0xFFFF0000), jnp.float32)
        hb = jnp.tanh(ub)
        logits = (lax.dot_general(h, Wfcf_ref[...], (((1,), (1,)), ((), ())),
                                  preferred_element_type=jnp.float32)
                  + lax.dot_general(hb, Wfcb_ref[...],
                                    (((1,), (1,)), ((), ())),
                                    preferred_element_type=jnp.float32)
                  + bfc_ref[...])
        m = jnp.max(logits, axis=1, keepdims=True)
        e = jnp.exp(logits - m)
        out_ref[...] = e / jnp.sum(e, axis=1, keepdims=True)


def _tc_rnn(u_tbh, W_hh_f, Wfcf, Wfcb, bfc):
    return pl.pallas_call(
        _rnn_body,
        grid=(NB, NT),
        in_specs=[
            pl.BlockSpec((TT, BB, H), lambda b, t: (t, b, 0)),
            pl.BlockSpec((H, H), lambda b, t: (0, 0)),
            pl.BlockSpec((O, H), lambda b, t: (0, 0)),
            pl.BlockSpec((O, H), lambda b, t: (0, 0)),
            pl.BlockSpec((1, O), lambda b, t: (0, 0)),
        ],
        out_specs=pl.BlockSpec((BB, O), lambda b, t: (b, 0)),
        out_shape=jax.ShapeDtypeStruct((B, O), jnp.float32),
        scratch_shapes=[pltpu.VMEM((BB, H), jnp.float32)],
        compiler_params=pltpu.CompilerParams(
            dimension_semantics=("parallel", "arbitrary")),
    )(u_tbh, W_hh_f, Wfcf, Wfcb, bfc)


def kernel(x, emb, W_ih_f, W_hh_f, b_ih_f, b_hh_f,
           W_ih_b, W_hh_b, b_ih_b, b_hh_b, W_fc, b_fc):
    bf = (b_ih_f + b_hh_f).reshape(1, H)
    bb = (b_ih_b + b_hh_b).reshape(1, H)
    p = _tc_project(emb, W_ih_f, bf, W_ih_b, bb)

    idx = jnp.transpose(x).reshape(-1)
    u = _sc_gather(p, idx)
    u_tbh = u.reshape(T, B, H)

    Wfcf = W_fc[:, :H]
    Wfcb = W_fc[:, H:]
    bfc = b_fc.reshape(1, O)

    return _tc_rnn(u_tbh, W_hh_f, Wfcf, Wfcb, bfc)
